# Initial kernel scaffold; baseline (speedup 1.0000x reference)
import functools

import jax
import jax.numpy as jnp
from jax import lax
from jax.experimental import pallas as pl
from jax.experimental.pallas import tpu as pltpu
from jax.experimental.pallas import tpu_sc as plsc

N = 10000
E = 320000
H = 128
C = 40

NC = 2
NS = 16
NL = 16
NW = NC * NS

K = 80
NB = 10
RB = N // NB

SQRT_E = 1.6487212707001282

_MESH = plsc.VectorSubcoreMesh(core_axis_name="c", subcore_axis_name="s")


def _zero16():
    return jnp.zeros((NL,), jnp.float32)


def _sc_pass1(xg, row, col):
    epw = E // NW
    nchunk = epw // K
    nrows = N // NS

    @functools.partial(
        pl.kernel,
        out_type=[
            jax.ShapeDtypeStruct((E,), jnp.float32),
            jax.ShapeDtypeStruct((E,), jnp.float32),
            jax.ShapeDtypeStruct((NC, N, NL), jnp.float32),
        ],
        mesh=_MESH,
        scratch_types=[
            pltpu.VMEM((K,), jnp.int32),
            pltpu.VMEM((K,), jnp.int32),
            pltpu.VMEM((K, H), jnp.float32),
            pltpu.VMEM((K, H), jnp.float32),
            pltpu.VMEM((K,), jnp.float32),
            pltpu.VMEM((K,), jnp.float32),
            pltpu.VMEM((K, NL), jnp.float32),
            pltpu.VMEM((NL,), jnp.float32),
            pltpu.VMEM((N // NS, NL), jnp.float32),
            pltpu.VMEM_SHARED((N, NL), jnp.float32),
            pltpu.SemaphoreType.DMA,
            pltpu.SemaphoreType.DMA,
        ],
    )
    def k(xg_hbm, row_hbm, col_hbm, a_hbm, b_hbm, sp_hbm,
          idxr, idxc, xr, xc, av, bv, srows, sqbuf, zbuf, ssh, sem1, sem2):
        cid = lax.axis_index("c")
        sid = lax.axis_index("s")
        wid = cid * NS + sid

        @pl.loop(0, nrows)
        def _(i):
            zbuf[i, :] = _zero16()

        @pl.loop(0, K)
        def _(i):
            srows[i, :] = _zero16()

        pltpu.sync_copy(zbuf, ssh.at[pl.ds(sid * nrows, nrows)])
        plsc.subcore_barrier()

        @pl.loop(0, nchunk)
        def _(ci):
            base = wid * epw + ci * K
            pltpu.sync_copy(row_hbm.at[pl.ds(base, K)], idxr)
            pltpu.sync_copy(col_hbm.at[pl.ds(base, K)], idxc)
            cp1 = pltpu.async_copy(xg_hbm.at[idxr], xr, sem1)
            cp2 = pltpu.async_copy(xg_hbm.at[idxc], xc, sem2)
            cp1.wait()
            cp2.wait()

            @pl.loop(0, K // NL)
            def _(g16):
                for j in range(NL):
                    jj = g16 * NL + j
                    acc = _zero16()
                    for d in range(H // NL):
                        t = (xr[jj, pl.ds(d * NL, NL)]
                             - xc[jj, pl.ds(d * NL, NL)])
                        acc = acc + t * t
                    sqbuf[j] = jnp.sum(acc)
                sq = sqbuf[...]
                pos = sq > 0.0
                sqv = jnp.where(pos, sq, 1.0)
                ii = plsc.bitcast(sqv, jnp.int32)
                ii = jnp.int32(0x5F3759DF) - (ii >> 1)
                y = plsc.bitcast(ii, jnp.float32)
                y = y * (1.5 - 0.5 * sqv * y * y)
                y = y * (1.5 - 0.5 * sqv * y * y)
                y = y * (1.5 - 0.5 * sqv * y * y)
                gn = jnp.where(pos, sqv * y, 0.0)
                s = 1.0 / (1.0 + jnp.exp(gn))
                ir = idxr[pl.ds(g16 * NL, NL)]
                ic = idxc[pl.ds(g16 * NL, NL)]
                msk = ir != ic
                a = jnp.where(msk, jnp.exp(s), 0.0)
                b = jnp.where(msk, jnp.exp(1.0 - s), 0.0)
                av[pl.ds(g16 * NL, NL)] = a
                bv[pl.ds(g16 * NL, NL)] = b
                ridx = g16 * NL + lax.iota(jnp.int32, NL)
                zidx = jnp.zeros((NL,), jnp.int32)
                plsc.store_scatter(srows, [ridx, zidx], a)
                plsc.store_scatter(srows, [ridx, zidx + 1], b)

            pltpu.sync_copy(av, a_hbm.at[pl.ds(base, K)])
            pltpu.sync_copy(bv, b_hbm.at[pl.ds(base, K)])
            pltpu.sync_copy(srows, ssh.at[idxr], add=True)

        plsc.subcore_barrier()
        pltpu.sync_copy(ssh.at[pl.ds(sid * nrows, nrows)],
                        sp_hbm.at[cid, pl.ds(sid * nrows, nrows)])

    return k(xg, row, col)


def _sc_pass2(yc, yd, a, b, row, col):
    epw = E // NS
    nchunk = epw // K
    nrows = N // NS

    @functools.partial(
        pl.kernel,
        out_type=jax.ShapeDtypeStruct((NC, N, H), jnp.float32),
        mesh=_MESH,
        scratch_types=[
            pltpu.VMEM((K,), jnp.int32),
            pltpu.VMEM((K,), jnp.int32),
            pltpu.VMEM((K,), jnp.float32),
            pltpu.VMEM((K, H), jnp.float32),
            pltpu.VMEM((N // NS, H), jnp.float32),
            pltpu.VMEM_SHARED((N, H), jnp.float32),
            pltpu.SemaphoreType.DMA,
        ],
    )
    def k(yc_hbm, yd_hbm, a_hbm, b_hbm, row_hbm, col_hbm, acc_hbm,
          idxr, idxc, wv, rows, zbuf, osh, sem):
        cid = lax.axis_index("c")
        sid = lax.axis_index("s")

        @pl.loop(0, nrows)
        def _(i):
            for d in range(H // NL):
                zbuf[i, pl.ds(d * NL, NL)] = _zero16()

        pltpu.sync_copy(zbuf, osh.at[pl.ds(sid * nrows, nrows)])
        plsc.subcore_barrier()

        def run(tab_hbm, w_hbm):
            @pl.loop(0, nchunk)
            def _(ci):
                base = sid * epw + ci * K
                pltpu.sync_copy(row_hbm.at[pl.ds(base, K)], idxr)
                pltpu.sync_copy(col_hbm.at[pl.ds(base, K)], idxc)
                pltpu.sync_copy(w_hbm.at[pl.ds(base, K)], wv)
                pltpu.async_copy(tab_hbm.at[idxr], rows, sem).wait()

                @pl.loop(0, K)
                def _(j):
                    sv = jnp.full((NL,), wv[j], jnp.float32)
                    for d in range(H // NL):
                        sl = pl.ds(d * NL, NL)
                        rows[j, sl] = rows[j, sl] * sv

                pltpu.sync_copy(rows, osh.at[idxc], add=True)

        @pl.when(cid == 0)
        def _():
            run(yc_hbm, a_hbm)

        @pl.when(cid == 1)
        def _():
            run(yd_hbm, b_hbm)

        plsc.subcore_barrier()
        pltpu.sync_copy(osh.at[pl.ds(sid * nrows, nrows)],
                        acc_hbm.at[cid, pl.ds(sid * nrows, nrows)])

    return k(yc, yd, a, b, row, col)


def _dot(x, w):
    return jnp.dot(x, w, preferred_element_type=jnp.float32)


def _full(shape):
    return pl.BlockSpec(shape, lambda i: tuple(0 for _ in shape))


def _rows(width):
    return pl.BlockSpec((RB, width), lambda i: (i, 0))


def _tc_pre(x, w1, b1, wg, wc, wd, ws, bs):
    def body(x_ref, w1_ref, b1_ref, wg_ref, wc_ref, wd_ref, ws_ref, bs_ref,
             h_ref, xg_ref, xc_ref, xd_ref, xs_ref):
        h = jnp.maximum(_dot(x_ref[...], w1_ref[...]) + b1_ref[...], 0.0)
        h_ref[...] = h
        xg_ref[...] = _dot(h, wg_ref[...])
        xc_ref[...] = _dot(h, wc_ref[...])
        xd_ref[...] = _dot(h, wd_ref[...])
        xs_ref[...] = _dot(h, ws_ref[...]) + bs_ref[...]

    o = jax.ShapeDtypeStruct((N, H), jnp.float32)
    return pl.pallas_call(
        body,
        grid=(NB,),
        in_specs=[_rows(H), _full((H, H)), _full((1, H)), _full((H, H)),
                  _full((H, H)), _full((H, H)), _full((H, H)), _full((1, H))],
        out_specs=[_rows(H)] * 5,
        out_shape=[o] * 5,
    )(x, w1, b1, wg, wc, wd, ws, bs)


def _tc_norm(xc, xd, sp):
    def body(xc_ref, xd_ref, sp_ref, yc_ref, yd_ref):
        sc = sp_ref[0, :, 0:1] + sp_ref[1, :, 0:1] + SQRT_E
        sd = sp_ref[0, :, 1:2] + sp_ref[1, :, 1:2] + SQRT_E
        yc_ref[...] = xc_ref[...] / sc
        yd_ref[...] = xd_ref[...] / sd

    o = jax.ShapeDtypeStruct((N, H), jnp.float32)
    return pl.pallas_call(
        body,
        grid=(NB,),
        in_specs=[_rows(H), _rows(H),
                  pl.BlockSpec((NC, RB, NL), lambda i: (0, i, 0))],
        out_specs=[_rows(H)] * 2,
        out_shape=[o] * 2,
    )(xc, xd, sp)


def _gate_combine(acc_ref, yc_ref, yd_ref, xs_ref, gc_ref, gd_ref, gs_ref,
                  gb_ref):
    oc = acc_ref[0] + SQRT_E * yc_ref[...]
    od = acc_ref[1] + SQRT_E * yd_ref[...]
    os_ = xs_ref[...]
    lg = (_dot(oc, gc_ref[...]) + _dot(od, gd_ref[...])
          + _dot(os_, gs_ref[...]) + gb_ref[...])
    m = jnp.max(lg, axis=1, keepdims=True)
    eg = jnp.exp(lg - m)
    gate = eg / jnp.sum(eg, axis=1, keepdims=True)
    return gate[:, 0:1] * oc + gate[:, 1:2] * od + gate[:, 2:3] * os_


def _tc_post0(acc, yc, yd, xs, hres, gc, gd, gs, gb, bns, bnsh,
              wg, wc, wd, ws, bs):
    def body(acc_ref, yc_ref, yd_ref, xs_ref, hres_ref, gc_ref, gd_ref,
             gs_ref, gb_ref, bns_ref, bnsh_ref, wg_ref, wc_ref, wd_ref,
             ws_ref, bs_ref, h_ref, xg_ref, xc_ref, xd_ref, xs2_ref):
        out = _gate_combine(acc_ref, yc_ref, yd_ref, xs_ref, gc_ref, gd_ref,
                            gs_ref, gb_ref)
        h = (jnp.maximum(out * bns_ref[...] + bnsh_ref[...], 0.0)
             + hres_ref[...])
        h_ref[...] = h
        xg_ref[...] = _dot(h, wg_ref[...])
        xc_ref[...] = _dot(h, wc_ref[...])
        xd_ref[...] = _dot(h, wd_ref[...])
        xs2_ref[...] = _dot(h, ws_ref[...]) + bs_ref[...]

    o = jax.ShapeDtypeStruct((N, H), jnp.float32)
    return pl.pallas_call(
        body,
        grid=(NB,),
        in_specs=[pl.BlockSpec((NC, RB, H), lambda i: (0, i, 0)),
                  _rows(H), _rows(H), _rows(H), _rows(H),
                  _full((H, 3)), _full((H, 3)), _full((H, 3)), _full((1, 3)),
                  _full((1, H)), _full((1, H)),
                  _full((H, H)), _full((H, H)), _full((H, H)), _full((H, H)),
                  _full((1, H))],
        out_specs=[_rows(H)] * 5,
        out_shape=[o] * 5,
    )(acc, yc, yd, xs, hres, gc, gd, gs, gb, bns, bnsh, wg, wc, wd, ws, bs)


def _tc_post1(acc, yc, yd, xs, hres, gc, gd, gs, gb, wcls, bcls):
    def body(acc_ref, yc_ref, yd_ref, xs_ref, hres_ref, gc_ref, gd_ref,
             gs_ref, gb_ref, wcls_ref, bcls_ref, o_ref):
        out = _gate_combine(acc_ref, yc_ref, yd_ref, xs_ref, gc_ref, gd_ref,
                            gs_ref, gb_ref)
        h = out + hres_ref[...]
        o_ref[...] = _dot(h, wcls_ref[...]) + bcls_ref[...]

    return pl.pallas_call(
        body,
        grid=(NB,),
        in_specs=[pl.BlockSpec((NC, RB, H), lambda i: (0, i, 0)),
                  _rows(H), _rows(H), _rows(H), _rows(H),
                  _full((H, 3)), _full((H, 3)), _full((H, 3)), _full((1, 3)),
                  _full((H, C)), _full((1, C))],
        out_specs=[_rows(C)],
        out_shape=[jax.ShapeDtypeStruct((N, C), jnp.float32)],
    )(acc, yc, yd, xs, hres, gc, gd, gs, gb, wcls, bcls)[0]


def _bn_fold(p):
    inv = 1.0 / jnp.sqrt(p['rv'] + 1e-5)
    scale = p['g'] * inv
    shift = p['b'] - p['rm'] * scale
    return scale, shift


def _prep_conv(p):
    gwt = p['gate_W'].T
    return dict(
        wg=p['W_g'].T, wc=p['W_con'].T, wd=p['W_dis'].T, ws=p['W_self'].T,
        bs=p['b_self'][None, :],
        gc=gwt[:H], gd=gwt[H:2 * H], gs=gwt[2 * H:], gb=p['gate_b'][None, :],
    )


def _conv_edge_phase(xg, xc, xd, row, col):
    a, b, sp = _sc_pass1(xg, row, col)
    yc, yd = _tc_norm(xc, xd, sp)
    acc = _sc_pass2(yc, yd, a, b, row, col)
    return acc, yc, yd


def kernel(x, edge_index, params):
    row = edge_index[0].astype(jnp.int32)
    col = edge_index[1].astype(jnp.int32)

    s0, sh0 = _bn_fold(params['in_bn'])
    w1 = params['in_W'].T * s0[None, :]
    b1 = (params['in_b'] * s0 + sh0)[None, :]
    c0 = _prep_conv(params['conv0'])
    c1 = _prep_conv(params['conv1'])
    bns0, bnsh0 = _bn_fold(params['bn0'])

    h0, xg0, xc0, xd0, xs0 = _tc_pre(x, w1, b1, c0['wg'], c0['wc'], c0['wd'],
                                     c0['ws'], c0['bs'])
    acc0, yc0, yd0 = _conv_edge_phase(xg0, xc0, xd0, row, col)
    h1, xg1, xc1, xd1, xs1 = _tc_post0(
        acc0, yc0, yd0, xs0, h0, c0['gc'], c0['gd'], c0['gs'], c0['gb'],
        bns0[None, :], bnsh0[None, :], c1['wg'], c1['wc'], c1['wd'], c1['ws'],
        c1['bs'])
    acc1, yc1, yd1 = _conv_edge_phase(xg1, xc1, xd1, row, col)
    return _tc_post1(acc1, yc1, yd1, xs1, h1, c1['gc'], c1['gd'], c1['gs'],
                     c1['gb'], params['cls_W'].T,
                     params['cls_b'][None, :])


# speedup vs baseline: 9.5717x; 9.5717x over previous
import dataclasses
import functools

import jax
import jax.numpy as jnp
from jax import lax
from jax.experimental import pallas as pl
from jax.experimental.pallas import tpu as pltpu
from jax.experimental.pallas import tpu_sc as plsc

N = 10000
E = 320000
H = 128
C = 40

NC = 2
NS = 16
NL = 16
NW = NC * NS

K = 80
NB = 10
RB = N // NB

SQRT_E = 1.6487212707001282

NP = 10240
RPT = NP // NS

_MESH = plsc.VectorSubcoreMesh(core_axis_name="c", subcore_axis_name="s")

_SC_PARAMS = pltpu.CompilerParams()
if "needs_layout_passes" in pltpu.CompilerParams.__dataclass_fields__:
    _SC_PARAMS = dataclasses.replace(_SC_PARAMS, needs_layout_passes=False)


def _zero16():
    return jnp.zeros((NL,), jnp.float32)


def _sc_pass1(xg, row, col):
    epw = E // NW
    nchunk = epw // K

    @functools.partial(
        pl.kernel,
        out_type=[
            jax.ShapeDtypeStruct((E,), jnp.float32),
            jax.ShapeDtypeStruct((E,), jnp.float32),
            jax.ShapeDtypeStruct((NC, NP, H), jnp.float32),
        ],
        mesh=_MESH,
        compiler_params=_SC_PARAMS,
        scratch_types=[
            pltpu.VMEM((K,), jnp.int32),
            pltpu.VMEM((K,), jnp.int32),
            pltpu.VMEM((K, H), jnp.float32),
            pltpu.VMEM((K, H), jnp.float32),
            pltpu.VMEM((K,), jnp.float32),
            pltpu.VMEM((K,), jnp.float32),
            pltpu.VMEM((K, H), jnp.float32),
            pltpu.VMEM_SHARED((NP, H), jnp.float32),
            pltpu.SemaphoreType.DMA,
            pltpu.SemaphoreType.DMA,
        ],
    )
    def k(xg_hbm, row_hbm, col_hbm, z_hbm, a_hbm, b_hbm, sp_hbm,
          idxr, idxc, xr, xc, av, bv, srows, ssh, sem1, sem2):
        cid = lax.axis_index("c")
        sid = lax.axis_index("s")
        wid = cid * NS + sid

        @pl.loop(0, K)
        def _(i):
            for d in range(H // NL):
                srows[i, pl.ds(d * NL, NL)] = _zero16()

        pltpu.sync_copy(z_hbm.at[pl.ds(sid * RPT, RPT)],
                        ssh.at[pl.ds(sid * RPT, RPT)])
        plsc.subcore_barrier()

        @pl.loop(0, nchunk)
        def _(ci):
            base = wid * epw + ci * K
            pltpu.sync_copy(row_hbm.at[pl.ds(base, K)], idxr)
            pltpu.sync_copy(col_hbm.at[pl.ds(base, K)], idxc)
            cp1 = pltpu.async_copy(xg_hbm.at[idxr], xr, sem1)
            cp2 = pltpu.async_copy(xg_hbm.at[idxc], xc, sem2)
            cp1.wait()
            cp2.wait()

            @pl.loop(0, K // NL)
            def _(g16):
                lane = lax.iota(jnp.int32, NL)
                sq = _zero16()
                for j in range(NL):
                    jj = g16 * NL + j
                    acc = _zero16()
                    for d in range(H // NL):
                        t = (xr[jj, pl.ds(d * NL, NL)]
                             - xc[jj, pl.ds(d * NL, NL)])
                        acc = acc + t * t
                    sj = jnp.full((NL,), jnp.sum(acc), jnp.float32)
                    sq = jnp.where(lane == j, sj, sq)
                pos = sq > 0.0
                sqv = jnp.where(pos, sq, 1.0)
                ii = plsc.bitcast(sqv, jnp.int32)
                ii = jnp.int32(0x5F3759DF) - (ii >> 1)
                y = plsc.bitcast(ii, jnp.float32)
                y = y * (1.5 - 0.5 * sqv * y * y)
                y = y * (1.5 - 0.5 * sqv * y * y)
                y = y * (1.5 - 0.5 * sqv * y * y)
                gn = jnp.where(pos, sqv * y, 0.0)
                s = 1.0 / (1.0 + jnp.exp(gn))
                ir = idxr[pl.ds(g16 * NL, NL)]
                ic = idxc[pl.ds(g16 * NL, NL)]
                msk = ir != ic
                a = jnp.where(msk, jnp.exp(s), 0.0)
                b = jnp.where(msk, jnp.exp(1.0 - s), 0.0)
                av[pl.ds(g16 * NL, NL)] = a
                bv[pl.ds(g16 * NL, NL)] = b
                ridx = g16 * NL + lax.iota(jnp.int32, NL)
                zidx = jnp.zeros((NL,), jnp.int32)
                plsc.store_scatter(srows, [ridx, zidx], a)
                plsc.store_scatter(srows, [ridx, zidx + 1], b)

            pltpu.sync_copy(av, a_hbm.at[pl.ds(base, K)])
            pltpu.sync_copy(bv, b_hbm.at[pl.ds(base, K)])
            pltpu.sync_copy(srows, ssh.at[idxr], add=True)

        plsc.subcore_barrier()
        pltpu.sync_copy(ssh.at[pl.ds(sid * RPT, RPT)],
                        sp_hbm.at[cid, pl.ds(sid * RPT, RPT)])

    return k(xg, row, col, jnp.zeros((NP, H), jnp.float32))


def _sc_pass2(yc, yd, a, b, row, col):
    epw = E // NS
    nchunk = epw // K

    @functools.partial(
        pl.kernel,
        out_type=jax.ShapeDtypeStruct((NC, NP, H), jnp.float32),
        mesh=_MESH,
        compiler_params=_SC_PARAMS,
        scratch_types=[
            pltpu.VMEM((K,), jnp.int32),
            pltpu.VMEM((K,), jnp.int32),
            pltpu.VMEM((K,), jnp.float32),
            pltpu.VMEM((K, H), jnp.float32),
            pltpu.VMEM_SHARED((NP, H), jnp.float32),
            pltpu.SemaphoreType.DMA,
        ],
    )
    def k(yc_hbm, yd_hbm, a_hbm, b_hbm, row_hbm, col_hbm, z_hbm, acc_hbm,
          idxr, idxc, wv, rows, osh, sem):
        cid = lax.axis_index("c")
        sid = lax.axis_index("s")

        pltpu.sync_copy(z_hbm.at[pl.ds(sid * RPT, RPT)],
                        osh.at[pl.ds(sid * RPT, RPT)])
        plsc.subcore_barrier()

        def run(tab_hbm, w_hbm):
            @pl.loop(0, nchunk)
            def _(ci):
                base = sid * epw + ci * K
                pltpu.sync_copy(row_hbm.at[pl.ds(base, K)], idxr)
                pltpu.sync_copy(col_hbm.at[pl.ds(base, K)], idxc)
                pltpu.sync_copy(w_hbm.at[pl.ds(base, K)], wv)
                pltpu.async_copy(tab_hbm.at[idxr], rows, sem).wait()

                @pl.loop(0, K // NL)
                def _(g16):
                    wg = wv[pl.ds(g16 * NL, NL)]
                    for j in range(NL):
                        jj = g16 * NL + j
                        sv = jnp.full((NL,), wg[j], jnp.float32)
                        for d in range(H // NL):
                            sl = pl.ds(d * NL, NL)
                            rows[jj, sl] = rows[jj, sl] * sv

                pltpu.sync_copy(rows, osh.at[idxc], add=True)

        @pl.when(cid == 0)
        def _():
            run(yc_hbm, a_hbm)

        @pl.when(cid == 1)
        def _():
            run(yd_hbm, b_hbm)

        plsc.subcore_barrier()
        pltpu.sync_copy(osh.at[pl.ds(sid * RPT, RPT)],
                        acc_hbm.at[cid, pl.ds(sid * RPT, RPT)])

    return k(yc, yd, a, b, row, col, jnp.zeros((NP, H), jnp.float32))


def _dot(x, w):
    return jnp.dot(x, w, preferred_element_type=jnp.float32)


def _full(shape):
    return pl.BlockSpec(shape, lambda i: tuple(0 for _ in shape))


def _rows(width):
    return pl.BlockSpec((RB, width), lambda i: (i, 0))


def _tc_pre(x, w1, b1, wg, wc, wd, ws, bs):
    def body(x_ref, w1_ref, b1_ref, wg_ref, wc_ref, wd_ref, ws_ref, bs_ref,
             h_ref, xg_ref, xc_ref, xd_ref, xs_ref):
        h = jnp.maximum(_dot(x_ref[...], w1_ref[...]) + b1_ref[...], 0.0)
        h_ref[...] = h
        xg_ref[...] = _dot(h, wg_ref[...])
        xc_ref[...] = _dot(h, wc_ref[...])
        xd_ref[...] = _dot(h, wd_ref[...])
        xs_ref[...] = _dot(h, ws_ref[...]) + bs_ref[...]

    o = jax.ShapeDtypeStruct((N, H), jnp.float32)
    return pl.pallas_call(
        body,
        grid=(NB,),
        in_specs=[_rows(H), _full((H, H)), _full((1, H)), _full((H, H)),
                  _full((H, H)), _full((H, H)), _full((H, H)), _full((1, H))],
        out_specs=[_rows(H)] * 5,
        out_shape=[o] * 5,
    )(x, w1, b1, wg, wc, wd, ws, bs)


def _tc_norm(xc, xd, sp):
    def body(xc_ref, xd_ref, sp_ref, yc_ref, yd_ref):
        sc = sp_ref[0, :, 0:1] + sp_ref[1, :, 0:1] + SQRT_E
        sd = sp_ref[0, :, 1:2] + sp_ref[1, :, 1:2] + SQRT_E
        yc_ref[...] = xc_ref[...] / sc
        yd_ref[...] = xd_ref[...] / sd

    o = jax.ShapeDtypeStruct((N, H), jnp.float32)
    return pl.pallas_call(
        body,
        grid=(NB,),
        in_specs=[_rows(H), _rows(H),
                  pl.BlockSpec((NC, RB, H), lambda i: (0, i, 0))],
        out_specs=[_rows(H)] * 2,
        out_shape=[o] * 2,
    )(xc, xd, sp)


def _gate_combine(acc_ref, yc_ref, yd_ref, xs_ref, gc_ref, gd_ref, gs_ref,
                  gb_ref):
    oc = acc_ref[0] + SQRT_E * yc_ref[...]
    od = acc_ref[1] + SQRT_E * yd_ref[...]
    os_ = xs_ref[...]
    lg = (_dot(oc, gc_ref[...]) + _dot(od, gd_ref[...])
          + _dot(os_, gs_ref[...]) + gb_ref[...])
    m = jnp.max(lg, axis=1, keepdims=True)
    eg = jnp.exp(lg - m)
    gate = eg / jnp.sum(eg, axis=1, keepdims=True)
    return gate[:, 0:1] * oc + gate[:, 1:2] * od + gate[:, 2:3] * os_


def _tc_post0(acc, yc, yd, xs, hres, gc, gd, gs, gb, bns, bnsh,
              wg, wc, wd, ws, bs):
    def body(acc_ref, yc_ref, yd_ref, xs_ref, hres_ref, gc_ref, gd_ref,
             gs_ref, gb_ref, bns_ref, bnsh_ref, wg_ref, wc_ref, wd_ref,
             ws_ref, bs_ref, h_ref, xg_ref, xc_ref, xd_ref, xs2_ref):
        out = _gate_combine(acc_ref, yc_ref, yd_ref, xs_ref, gc_ref, gd_ref,
                            gs_ref, gb_ref)
        h = (jnp.maximum(out * bns_ref[...] + bnsh_ref[...], 0.0)
             + hres_ref[...])
        h_ref[...] = h
        xg_ref[...] = _dot(h, wg_ref[...])
        xc_ref[...] = _dot(h, wc_ref[...])
        xd_ref[...] = _dot(h, wd_ref[...])
        xs2_ref[...] = _dot(h, ws_ref[...]) + bs_ref[...]

    o = jax.ShapeDtypeStruct((N, H), jnp.float32)
    return pl.pallas_call(
        body,
        grid=(NB,),
        in_specs=[pl.BlockSpec((NC, RB, H), lambda i: (0, i, 0)),
                  _rows(H), _rows(H), _rows(H), _rows(H),
                  _full((H, 3)), _full((H, 3)), _full((H, 3)), _full((1, 3)),
                  _full((1, H)), _full((1, H)),
                  _full((H, H)), _full((H, H)), _full((H, H)), _full((H, H)),
                  _full((1, H))],
        out_specs=[_rows(H)] * 5,
        out_shape=[o] * 5,
    )(acc, yc, yd, xs, hres, gc, gd, gs, gb, bns, bnsh, wg, wc, wd, ws, bs)


def _tc_post1(acc, yc, yd, xs, hres, gc, gd, gs, gb, wcls, bcls):
    def body(acc_ref, yc_ref, yd_ref, xs_ref, hres_ref, gc_ref, gd_ref,
             gs_ref, gb_ref, wcls_ref, bcls_ref, o_ref):
        out = _gate_combine(acc_ref, yc_ref, yd_ref, xs_ref, gc_ref, gd_ref,
                            gs_ref, gb_ref)
        h = out + hres_ref[...]
        o_ref[...] = _dot(h, wcls_ref[...]) + bcls_ref[...]

    return pl.pallas_call(
        body,
        grid=(NB,),
        in_specs=[pl.BlockSpec((NC, RB, H), lambda i: (0, i, 0)),
                  _rows(H), _rows(H), _rows(H), _rows(H),
                  _full((H, 3)), _full((H, 3)), _full((H, 3)), _full((1, 3)),
                  _full((H, C)), _full((1, C))],
        out_specs=[_rows(C)],
        out_shape=[jax.ShapeDtypeStruct((N, C), jnp.float32)],
    )(acc, yc, yd, xs, hres, gc, gd, gs, gb, wcls, bcls)[0]


def _bn_fold(p):
    inv = 1.0 / jnp.sqrt(p['rv'] + 1e-5)
    scale = p['g'] * inv
    shift = p['b'] - p['rm'] * scale
    return scale, shift


def _prep_conv(p):
    gwt = p['gate_W'].T
    return dict(
        wg=p['W_g'].T, wc=p['W_con'].T, wd=p['W_dis'].T, ws=p['W_self'].T,
        bs=p['b_self'][None, :],
        gc=gwt[:H], gd=gwt[H:2 * H], gs=gwt[2 * H:], gb=p['gate_b'][None, :],
    )


def _conv_edge_phase(xg, xc, xd, row, col):
    a, b, sp = _sc_pass1(xg, row, col)
    yc, yd = _tc_norm(xc, xd, sp)
    acc = _sc_pass2(yc, yd, a, b, row, col)
    return acc, yc, yd


def kernel(x, edge_index, params):
    row = edge_index[0].astype(jnp.int32)
    col = edge_index[1].astype(jnp.int32)

    s0, sh0 = _bn_fold(params['in_bn'])
    w1 = params['in_W'].T * s0[None, :]
    b1 = (params['in_b'] * s0 + sh0)[None, :]
    c0 = _prep_conv(params['conv0'])
    c1 = _prep_conv(params['conv1'])
    bns0, bnsh0 = _bn_fold(params['bn0'])

    h0, xg0, xc0, xd0, xs0 = _tc_pre(x, w1, b1, c0['wg'], c0['wc'], c0['wd'],
                                     c0['ws'], c0['bs'])
    acc0, yc0, yd0 = _conv_edge_phase(xg0, xc0, xd0, row, col)
    h1, xg1, xc1, xd1, xs1 = _tc_post0(
        acc0, yc0, yd0, xs0, h0, c0['gc'], c0['gd'], c0['gs'], c0['gb'],
        bns0[None, :], bnsh0[None, :], c1['wg'], c1['wc'], c1['wd'], c1['ws'],
        c1['bs'])
    acc1, yc1, yd1 = _conv_edge_phase(xg1, xc1, xd1, row, col)
    return _tc_post1(acc1, yc1, yd1, xs1, h1, c1['gc'], c1['gd'], c1['gs'],
                     c1['gb'], params['cls_W'].T,
                     params['cls_b'][None, :])


# speedup vs baseline: 12.1336x; 1.2676x over previous
import dataclasses
import functools

import jax
import jax.numpy as jnp
from jax import lax
from jax.experimental import pallas as pl
from jax.experimental.pallas import tpu as pltpu
from jax.experimental.pallas import tpu_sc as plsc

N = 10000
E = 320000
H = 128
C = 40

NC = 2
NS = 16
NL = 16
NW = NC * NS

K = 80
NB = 10
RB = N // NB

SQRT_E = 1.6487212707001282

NP = 10240
RPT = NP // NS

_MESH = plsc.VectorSubcoreMesh(core_axis_name="c", subcore_axis_name="s")

_SC_PARAMS = pltpu.CompilerParams()
if "needs_layout_passes" in pltpu.CompilerParams.__dataclass_fields__:
    _SC_PARAMS = dataclasses.replace(_SC_PARAMS, needs_layout_passes=False)


def _zero16():
    return jnp.zeros((NL,), jnp.float32)


def _sc_pass1(xg, row, col):
    epw = E // NW
    nchunk = epw // K

    @functools.partial(
        pl.kernel,
        out_type=[
            jax.ShapeDtypeStruct((E,), jnp.float32),
            jax.ShapeDtypeStruct((E,), jnp.float32),
            jax.ShapeDtypeStruct((NC, NP, H), jnp.float32),
        ],
        mesh=_MESH,
        compiler_params=_SC_PARAMS,
        scratch_types=[
            pltpu.VMEM((K,), jnp.int32),
            pltpu.VMEM((K,), jnp.int32),
            pltpu.VMEM((K, H), jnp.float32),
            pltpu.VMEM((K, H), jnp.float32),
            pltpu.VMEM((K,), jnp.float32),
            pltpu.VMEM((K,), jnp.float32),
            pltpu.VMEM((K, H), jnp.float32),
            pltpu.VMEM_SHARED((NP, H), jnp.float32),
            pltpu.SemaphoreType.DMA,
            pltpu.SemaphoreType.DMA,
        ],
    )
    def k(xg_hbm, row_hbm, col_hbm, z_hbm, a_hbm, b_hbm, sp_hbm,
          idxr, idxc, xr, xc, av, bv, srows, ssh, sem1, sem2):
        cid = lax.axis_index("c")
        sid = lax.axis_index("s")
        wid = cid * NS + sid

        @pl.loop(0, K)
        def _(i):
            for d in range(H // NL):
                srows[i, pl.ds(d * NL, NL)] = _zero16()

        pltpu.sync_copy(z_hbm.at[pl.ds(sid * RPT, RPT)],
                        ssh.at[pl.ds(sid * RPT, RPT)])
        plsc.subcore_barrier()

        @pl.loop(0, nchunk)
        def _(ci):
            base = wid * epw + ci * K
            pltpu.sync_copy(row_hbm.at[pl.ds(base, K)], idxr)
            pltpu.sync_copy(col_hbm.at[pl.ds(base, K)], idxc)
            cp1 = pltpu.async_copy(xg_hbm.at[idxr], xr, sem1)
            cp2 = pltpu.async_copy(xg_hbm.at[idxc], xc, sem2)
            cp1.wait()
            cp2.wait()

            @pl.loop(0, K // NL)
            def _(g16):
                lane = lax.iota(jnp.int32, NL)
                sq = _zero16()
                for j in range(NL):
                    jj = g16 * NL + j
                    acc = _zero16()
                    for d in range(H // NL):
                        t = (xr[jj, pl.ds(d * NL, NL)]
                             - xc[jj, pl.ds(d * NL, NL)])
                        acc = acc + t * t
                    sj = jnp.full((NL,), jnp.sum(acc), jnp.float32)
                    sq = jnp.where(lane == j, sj, sq)
                pos = sq > 0.0
                sqv = jnp.where(pos, sq, 1.0)
                ii = plsc.bitcast(sqv, jnp.int32)
                ii = jnp.int32(0x5F3759DF) - (ii >> 1)
                y = plsc.bitcast(ii, jnp.float32)
                y = y * (1.5 - 0.5 * sqv * y * y)
                y = y * (1.5 - 0.5 * sqv * y * y)
                y = y * (1.5 - 0.5 * sqv * y * y)
                gn = jnp.where(pos, sqv * y, 0.0)
                s = 1.0 / (1.0 + jnp.exp(gn))
                ir = idxr[pl.ds(g16 * NL, NL)]
                ic = idxc[pl.ds(g16 * NL, NL)]
                msk = ir != ic
                a = jnp.where(msk, jnp.exp(s), 0.0)
                b = jnp.where(msk, jnp.exp(1.0 - s), 0.0)
                av[pl.ds(g16 * NL, NL)] = a
                bv[pl.ds(g16 * NL, NL)] = b
                ridx = g16 * NL + lax.iota(jnp.int32, NL)
                zidx = jnp.zeros((NL,), jnp.int32)
                plsc.store_scatter(srows, [ridx, zidx], a)
                plsc.store_scatter(srows, [ridx, zidx + 1], b)

            pltpu.sync_copy(av, a_hbm.at[pl.ds(base, K)])
            pltpu.sync_copy(bv, b_hbm.at[pl.ds(base, K)])
            pltpu.sync_copy(srows, ssh.at[idxr], add=True)

        plsc.subcore_barrier()
        pltpu.sync_copy(ssh.at[pl.ds(sid * RPT, RPT)],
                        sp_hbm.at[cid, pl.ds(sid * RPT, RPT)])

    return k(xg, row, col, jnp.zeros((NP, H), jnp.float32))


def _sc_pass2(yc, yd, rcpack, abpack):
    nchunk = (E // K) // NS

    @functools.partial(
        pl.kernel,
        out_type=jax.ShapeDtypeStruct((NC, NP, H), jnp.float32),
        mesh=_MESH,
        compiler_params=_SC_PARAMS,
        scratch_types=[
            pltpu.VMEM((2, 2, K), jnp.int32),
            pltpu.VMEM((2, 2, K), jnp.int32),
            pltpu.VMEM((2, K, H), jnp.float32),
            pltpu.VMEM_SHARED((NP, H), jnp.float32),
            pltpu.SemaphoreType.DMA,
            pltpu.SemaphoreType.DMA,
            pltpu.SemaphoreType.DMA,
            pltpu.SemaphoreType.DMA,
        ],
    )
    def k(yc_hbm, yd_hbm, rc_hbm, ab_hbm, z_hbm, acc_hbm,
          rcb, abb, rows, osh, semL0, semL1, semG0, semG1):
        cid = lax.axis_index("c")
        sid = lax.axis_index("s")
        c0 = sid * nchunk

        pltpu.sync_copy(z_hbm.at[pl.ds(sid * RPT, RPT)],
                        osh.at[pl.ds(sid * RPT, RPT)])
        plsc.subcore_barrier()

        semL = (semL0, semL1)
        semG = (semG0, semG1)

        def issue_load(i, b):
            pltpu.async_copy(rc_hbm.at[c0 + i], rcb.at[b], semL[b])
            pltpu.async_copy(ab_hbm.at[c0 + i], abb.at[b], semL[b])

        def wait_load(b):
            pltpu.make_async_copy(rc_hbm.at[c0], rcb.at[b], semL[b]).wait()
            pltpu.make_async_copy(ab_hbm.at[c0], abb.at[b], semL[b]).wait()

        def run(tab_hbm, wsel):
            def issue_gather(b):
                pltpu.async_copy(tab_hbm.at[rcb.at[b, 0]], rows.at[b],
                                 semG[b])

            def wait_gather(b):
                pltpu.make_async_copy(tab_hbm.at[rcb.at[b, 0]], rows.at[b],
                                      semG[b]).wait()

            def compute(b):
                @pl.loop(0, K // NL)
                def _(g16):
                    wbits = abb[b, wsel, pl.ds(g16 * NL, NL)]
                    wg = plsc.bitcast(wbits, jnp.float32)
                    for j in range(NL):
                        jj = g16 * NL + j
                        sv = jnp.full((NL,), wg[j], jnp.float32)
                        for d in range(H // NL):
                            sl = pl.ds(d * NL, NL)
                            rows[b, jj, sl] = rows[b, jj, sl] * sv

                pltpu.sync_copy(rows.at[b], osh.at[rcb.at[b, 1]], add=True)

            issue_load(0, 0)
            issue_load(1, 1)
            wait_load(0)
            issue_gather(0)

            @pl.loop(0, nchunk // 2)
            def _(j):
                i = 2 * j
                wait_gather(0)
                compute(0)

                @pl.when(i + 2 < nchunk)
                def _():
                    issue_load(i + 2, 0)

                wait_load(1)
                issue_gather(1)

                i2 = i + 1
                wait_gather(1)
                compute(1)

                @pl.when(i2 + 2 < nchunk)
                def _():
                    issue_load(i2 + 2, 1)

                @pl.when(i2 + 1 < nchunk)
                def _():
                    wait_load(0)
                    issue_gather(0)

        @pl.when(cid == 0)
        def _():
            run(yc_hbm, 0)

        @pl.when(cid == 1)
        def _():
            run(yd_hbm, 1)

        plsc.subcore_barrier()
        pltpu.sync_copy(osh.at[pl.ds(sid * RPT, RPT)],
                        acc_hbm.at[cid, pl.ds(sid * RPT, RPT)])

    return k(yc, yd, rcpack, abpack, jnp.zeros((NP, H), jnp.float32))


def _dot(x, w):
    return jnp.dot(x, w, preferred_element_type=jnp.float32)


def _full(shape):
    return pl.BlockSpec(shape, lambda i: tuple(0 for _ in shape))


def _rows(width):
    return pl.BlockSpec((RB, width), lambda i: (i, 0))


def _tc_pre(x, w1, b1, wg, wc, wd, ws, bs):
    def body(x_ref, w1_ref, b1_ref, wg_ref, wc_ref, wd_ref, ws_ref, bs_ref,
             h_ref, xg_ref, xc_ref, xd_ref, xs_ref):
        h = jnp.maximum(_dot(x_ref[...], w1_ref[...]) + b1_ref[...], 0.0)
        h_ref[...] = h
        xg_ref[...] = _dot(h, wg_ref[...])
        xc_ref[...] = _dot(h, wc_ref[...])
        xd_ref[...] = _dot(h, wd_ref[...])
        xs_ref[...] = _dot(h, ws_ref[...]) + bs_ref[...]

    o = jax.ShapeDtypeStruct((N, H), jnp.float32)
    return pl.pallas_call(
        body,
        grid=(NB,),
        in_specs=[_rows(H), _full((H, H)), _full((1, H)), _full((H, H)),
                  _full((H, H)), _full((H, H)), _full((H, H)), _full((1, H))],
        out_specs=[_rows(H)] * 5,
        out_shape=[o] * 5,
    )(x, w1, b1, wg, wc, wd, ws, bs)


def _tc_norm(xc, xd, sp):
    def body(xc_ref, xd_ref, sp_ref, yc_ref, yd_ref):
        sc = sp_ref[0, :, 0:1] + sp_ref[1, :, 0:1] + SQRT_E
        sd = sp_ref[0, :, 1:2] + sp_ref[1, :, 1:2] + SQRT_E
        yc_ref[...] = xc_ref[...] / sc
        yd_ref[...] = xd_ref[...] / sd

    o = jax.ShapeDtypeStruct((N, H), jnp.float32)
    return pl.pallas_call(
        body,
        grid=(NB,),
        in_specs=[_rows(H), _rows(H),
                  pl.BlockSpec((NC, RB, H), lambda i: (0, i, 0))],
        out_specs=[_rows(H)] * 2,
        out_shape=[o] * 2,
    )(xc, xd, sp)


def _gate_combine(acc_ref, yc_ref, yd_ref, xs_ref, gc_ref, gd_ref, gs_ref,
                  gb_ref):
    oc = acc_ref[0] + SQRT_E * yc_ref[...]
    od = acc_ref[1] + SQRT_E * yd_ref[...]
    os_ = xs_ref[...]
    lg = (_dot(oc, gc_ref[...]) + _dot(od, gd_ref[...])
          + _dot(os_, gs_ref[...]) + gb_ref[...])
    m = jnp.max(lg, axis=1, keepdims=True)
    eg = jnp.exp(lg - m)
    gate = eg / jnp.sum(eg, axis=1, keepdims=True)
    return gate[:, 0:1] * oc + gate[:, 1:2] * od + gate[:, 2:3] * os_


def _tc_post0(acc, yc, yd, xs, hres, gc, gd, gs, gb, bns, bnsh,
              wg, wc, wd, ws, bs):
    def body(acc_ref, yc_ref, yd_ref, xs_ref, hres_ref, gc_ref, gd_ref,
             gs_ref, gb_ref, bns_ref, bnsh_ref, wg_ref, wc_ref, wd_ref,
             ws_ref, bs_ref, h_ref, xg_ref, xc_ref, xd_ref, xs2_ref):
        out = _gate_combine(acc_ref, yc_ref, yd_ref, xs_ref, gc_ref, gd_ref,
                            gs_ref, gb_ref)
        h = (jnp.maximum(out * bns_ref[...] + bnsh_ref[...], 0.0)
             + hres_ref[...])
        h_ref[...] = h
        xg_ref[...] = _dot(h, wg_ref[...])
        xc_ref[...] = _dot(h, wc_ref[...])
        xd_ref[...] = _dot(h, wd_ref[...])
        xs2_ref[...] = _dot(h, ws_ref[...]) + bs_ref[...]

    o = jax.ShapeDtypeStruct((N, H), jnp.float32)
    return pl.pallas_call(
        body,
        grid=(NB,),
        in_specs=[pl.BlockSpec((NC, RB, H), lambda i: (0, i, 0)),
                  _rows(H), _rows(H), _rows(H), _rows(H),
                  _full((H, 3)), _full((H, 3)), _full((H, 3)), _full((1, 3)),
                  _full((1, H)), _full((1, H)),
                  _full((H, H)), _full((H, H)), _full((H, H)), _full((H, H)),
                  _full((1, H))],
        out_specs=[_rows(H)] * 5,
        out_shape=[o] * 5,
    )(acc, yc, yd, xs, hres, gc, gd, gs, gb, bns, bnsh, wg, wc, wd, ws, bs)


def _tc_post1(acc, yc, yd, xs, hres, gc, gd, gs, gb, wcls, bcls):
    def body(acc_ref, yc_ref, yd_ref, xs_ref, hres_ref, gc_ref, gd_ref,
             gs_ref, gb_ref, wcls_ref, bcls_ref, o_ref):
        out = _gate_combine(acc_ref, yc_ref, yd_ref, xs_ref, gc_ref, gd_ref,
                            gs_ref, gb_ref)
        h = out + hres_ref[...]
        o_ref[...] = _dot(h, wcls_ref[...]) + bcls_ref[...]

    return pl.pallas_call(
        body,
        grid=(NB,),
        in_specs=[pl.BlockSpec((NC, RB, H), lambda i: (0, i, 0)),
                  _rows(H), _rows(H), _rows(H), _rows(H),
                  _full((H, 3)), _full((H, 3)), _full((H, 3)), _full((1, 3)),
                  _full((H, C)), _full((1, C))],
        out_specs=[_rows(C)],
        out_shape=[jax.ShapeDtypeStruct((N, C), jnp.float32)],
    )(acc, yc, yd, xs, hres, gc, gd, gs, gb, wcls, bcls)[0]


def _bn_fold(p):
    inv = 1.0 / jnp.sqrt(p['rv'] + 1e-5)
    scale = p['g'] * inv
    shift = p['b'] - p['rm'] * scale
    return scale, shift


def _prep_conv(p):
    gwt = p['gate_W'].T
    return dict(
        wg=p['W_g'].T, wc=p['W_con'].T, wd=p['W_dis'].T, ws=p['W_self'].T,
        bs=p['b_self'][None, :],
        gc=gwt[:H], gd=gwt[H:2 * H], gs=gwt[2 * H:], gb=p['gate_b'][None, :],
    )


def _conv_edge_phase(xg, xc, xd, row, col, rcpack):
    a, b, sp = _sc_pass1(xg, row, col)
    yc, yd = _tc_norm(xc, xd, sp)
    nch = E // K
    abpack = jnp.stack(
        [lax.bitcast_convert_type(a, jnp.int32).reshape(nch, K),
         lax.bitcast_convert_type(b, jnp.int32).reshape(nch, K)], axis=1)
    acc = _sc_pass2(yc, yd, rcpack, abpack)
    return acc, yc, yd


def kernel(x, edge_index, params):
    row = edge_index[0].astype(jnp.int32)
    col = edge_index[1].astype(jnp.int32)

    s0, sh0 = _bn_fold(params['in_bn'])
    w1 = params['in_W'].T * s0[None, :]
    b1 = (params['in_b'] * s0 + sh0)[None, :]
    c0 = _prep_conv(params['conv0'])
    c1 = _prep_conv(params['conv1'])
    bns0, bnsh0 = _bn_fold(params['bn0'])

    nch = E // K
    rcpack = jnp.stack([row.reshape(nch, K), col.reshape(nch, K)], axis=1)

    h0, xg0, xc0, xd0, xs0 = _tc_pre(x, w1, b1, c0['wg'], c0['wc'], c0['wd'],
                                     c0['ws'], c0['bs'])
    acc0, yc0, yd0 = _conv_edge_phase(xg0, xc0, xd0, row, col, rcpack)
    h1, xg1, xc1, xd1, xs1 = _tc_post0(
        acc0, yc0, yd0, xs0, h0, c0['gc'], c0['gd'], c0['gs'], c0['gb'],
        bns0[None, :], bnsh0[None, :], c1['wg'], c1['wc'], c1['wd'], c1['ws'],
        c1['bs'])
    acc1, yc1, yd1 = _conv_edge_phase(xg1, xc1, xd1, row, col, rcpack)
    return _tc_post1(acc1, yc1, yd1, xs1, h1, c1['gc'], c1['gd'], c1['gs'],
                     c1['gb'], params['cls_W'].T,
                     params['cls_b'][None, :])


# speedup vs baseline: 13.8632x; 1.1425x over previous
import dataclasses
import functools

import jax
import jax.numpy as jnp
from jax import lax
from jax.experimental import pallas as pl
from jax.experimental.pallas import tpu as pltpu
from jax.experimental.pallas import tpu_sc as plsc

N = 10000
E = 320000
H = 128
C = 40

NC = 2
NS = 16
NL = 16
NW = NC * NS

K = 80
NB = 10
RB = N // NB

SQRT_E = 1.6487212707001282

NP = 10240
RPT = NP // NS

_MESH = plsc.VectorSubcoreMesh(core_axis_name="c", subcore_axis_name="s")

_SC_PARAMS = pltpu.CompilerParams()
if "needs_layout_passes" in pltpu.CompilerParams.__dataclass_fields__:
    _SC_PARAMS = dataclasses.replace(_SC_PARAMS, needs_layout_passes=False)


def _zero16():
    return jnp.zeros((NL,), jnp.float32)


def _sc_pass1(xg, rcpack):
    nchunk = (E // K) // NW
    nloop = nchunk // 2

    @functools.partial(
        pl.kernel,
        out_type=[
            jax.ShapeDtypeStruct((E // K, 2, K), jnp.float32),
            jax.ShapeDtypeStruct((NC, NS, 2, NP), jnp.float32),
        ],
        mesh=_MESH,
        compiler_params=_SC_PARAMS,
        scratch_types=[
            pltpu.VMEM((2, 2, K), jnp.int32),
            pltpu.VMEM((2, K, H), jnp.float32),
            pltpu.VMEM((2, K, H), jnp.float32),
            pltpu.VMEM((2, 2, K), jnp.float32),
            pltpu.VMEM((2, NP), jnp.float32),
            pltpu.SemaphoreType.DMA,
            pltpu.SemaphoreType.DMA,
            pltpu.SemaphoreType.DMA,
            pltpu.SemaphoreType.DMA,
            pltpu.SemaphoreType.DMA,
            pltpu.SemaphoreType.DMA,
        ],
    )
    def k(xg_hbm, rc_hbm, ab_hbm, sp_hbm,
          rcb, xr, xc, avbv, sab,
          semL0, semL1, semG0, semG1, semS0, semS1):
        cid = lax.axis_index("c")
        sid = lax.axis_index("s")
        wid = cid * NS + sid
        c0 = wid * nchunk

        @pl.loop(0, NP // NL)
        def _(i):
            sab[0, pl.ds(i * NL, NL)] = _zero16()
            sab[1, pl.ds(i * NL, NL)] = _zero16()

        semL = (semL0, semL1)
        semG = (semG0, semG1)
        semS = (semS0, semS1)

        def issue_load(i, b):
            pltpu.async_copy(rc_hbm.at[c0 + i], rcb.at[b], semL[b])

        def wait_load(b):
            pltpu.make_async_copy(rc_hbm.at[c0], rcb.at[b], semL[b]).wait()

        def issue_gather(b):
            pltpu.async_copy(xg_hbm.at[rcb.at[b, 0]], xr.at[b], semG[b])
            pltpu.async_copy(xg_hbm.at[rcb.at[b, 1]], xc.at[b], semG[b])

        def wait_gather(b):
            pltpu.make_async_copy(xg_hbm.at[rcb.at[b, 0]], xr.at[b],
                                  semG[b]).wait()
            pltpu.make_async_copy(xg_hbm.at[rcb.at[b, 1]], xc.at[b],
                                  semG[b]).wait()

        def issue_store(i, b):
            pltpu.async_copy(avbv.at[b], ab_hbm.at[c0 + i], semS[b])

        def wait_store(b):
            pltpu.make_async_copy(avbv.at[b], ab_hbm.at[c0], semS[b]).wait()

        def compute(b):
            @pl.loop(0, K // NL)
            def _(g16):
                lane = lax.iota(jnp.int32, NL)
                sq = _zero16()
                for j in range(NL):
                    jj = g16 * NL + j
                    acc = _zero16()
                    for d in range(H // NL):
                        t = (xr[b, jj, pl.ds(d * NL, NL)]
                             - xc[b, jj, pl.ds(d * NL, NL)])
                        acc = acc + t * t
                    sj = jnp.full((NL,), jnp.sum(acc), jnp.float32)
                    sq = jnp.where(lane == j, sj, sq)
                pos = sq > 0.0
                sqv = jnp.where(pos, sq, 1.0)
                ii = plsc.bitcast(sqv, jnp.int32)
                ii = jnp.int32(0x5F3759DF) - (ii >> 1)
                y = plsc.bitcast(ii, jnp.float32)
                y = y * (1.5 - 0.5 * sqv * y * y)
                y = y * (1.5 - 0.5 * sqv * y * y)
                y = y * (1.5 - 0.5 * sqv * y * y)
                gn = jnp.where(pos, sqv * y, 0.0)
                s = 1.0 / (1.0 + jnp.exp(gn))
                ir = rcb[b, 0, pl.ds(g16 * NL, NL)]
                ic = rcb[b, 1, pl.ds(g16 * NL, NL)]
                msk = ir != ic
                a = jnp.where(msk, jnp.exp(s), 0.0)
                bw = jnp.where(msk, jnp.exp(1.0 - s), 0.0)
                avbv[b, 0, pl.ds(g16 * NL, NL)] = a
                avbv[b, 1, pl.ds(g16 * NL, NL)] = bw
                tz = jnp.zeros((NL,), jnp.int32)
                for j in range(NL):
                    lm = lane == j
                    plsc.addupdate_scatter(sab, [tz, ir], a, mask=lm)
                    plsc.addupdate_scatter(sab, [tz + 1, ir], bw, mask=lm)

        issue_load(0, 0)
        issue_load(1, 1)
        wait_load(0)
        issue_gather(0)

        @pl.loop(0, nloop)
        def _(j):
            i = 2 * j
            wait_gather(0)

            @pl.when(i >= 2)
            def _():
                wait_store(0)

            compute(0)
            issue_store(i, 0)
            issue_load(i + 2, 0)
            wait_load(1)
            issue_gather(1)

            i2 = i + 1
            wait_gather(1)

            @pl.when(i2 >= 2)
            def _():
                wait_store(1)

            compute(1)
            issue_store(i2, 1)

            @pl.when(i2 + 2 < nchunk)
            def _():
                issue_load(i2 + 2, 1)

            wait_load(0)
            issue_gather(0)

        wait_gather(0)
        wait_store(0)
        compute(0)
        issue_store(nchunk - 1, 0)
        wait_store(1)
        wait_store(0)

        pltpu.sync_copy(sab, sp_hbm.at[cid, sid])

    return k(xg, rcpack)


def _sc_pass2(yc, yd, rcpack, abpack):
    nchunk = (E // K) // NS

    @functools.partial(
        pl.kernel,
        out_type=jax.ShapeDtypeStruct((NC, NP, H), jnp.float32),
        mesh=_MESH,
        compiler_params=_SC_PARAMS,
        scratch_types=[
            pltpu.VMEM((2, 2, K), jnp.int32),
            pltpu.VMEM((2, 2, K), jnp.int32),
            pltpu.VMEM((2, K, H), jnp.float32),
            pltpu.VMEM_SHARED((NP, H), jnp.float32),
            pltpu.SemaphoreType.DMA,
            pltpu.SemaphoreType.DMA,
            pltpu.SemaphoreType.DMA,
            pltpu.SemaphoreType.DMA,
        ],
    )
    def k(yc_hbm, yd_hbm, rc_hbm, ab_hbm, z_hbm, acc_hbm,
          rcb, abb, rows, osh, semL0, semL1, semG0, semG1):
        cid = lax.axis_index("c")
        sid = lax.axis_index("s")
        c0 = sid * nchunk

        pltpu.sync_copy(z_hbm.at[pl.ds(sid * RPT, RPT)],
                        osh.at[pl.ds(sid * RPT, RPT)])
        plsc.subcore_barrier()

        semL = (semL0, semL1)
        semG = (semG0, semG1)

        def issue_load(i, b):
            pltpu.async_copy(rc_hbm.at[c0 + i], rcb.at[b], semL[b])
            pltpu.async_copy(ab_hbm.at[c0 + i], abb.at[b], semL[b])

        def wait_load(b):
            pltpu.make_async_copy(rc_hbm.at[c0], rcb.at[b], semL[b]).wait()
            pltpu.make_async_copy(ab_hbm.at[c0], abb.at[b], semL[b]).wait()

        def run(tab_hbm, wsel):
            def issue_gather(b):
                pltpu.async_copy(tab_hbm.at[rcb.at[b, 0]], rows.at[b],
                                 semG[b])

            def wait_gather(b):
                pltpu.make_async_copy(tab_hbm.at[rcb.at[b, 0]], rows.at[b],
                                      semG[b]).wait()

            def compute(b):
                @pl.loop(0, K // NL)
                def _(g16):
                    wbits = abb[b, wsel, pl.ds(g16 * NL, NL)]
                    wg = plsc.bitcast(wbits, jnp.float32)
                    for j in range(NL):
                        jj = g16 * NL + j
                        sv = jnp.full((NL,), wg[j], jnp.float32)
                        for d in range(H // NL):
                            sl = pl.ds(d * NL, NL)
                            rows[b, jj, sl] = rows[b, jj, sl] * sv

                pltpu.sync_copy(rows.at[b], osh.at[rcb.at[b, 1]], add=True)

            issue_load(0, 0)
            issue_load(1, 1)
            wait_load(0)
            issue_gather(0)

            @pl.loop(0, nchunk // 2)
            def _(j):
                i = 2 * j
                wait_gather(0)
                compute(0)

                @pl.when(i + 2 < nchunk)
                def _():
                    issue_load(i + 2, 0)

                wait_load(1)
                issue_gather(1)

                i2 = i + 1
                wait_gather(1)
                compute(1)

                @pl.when(i2 + 2 < nchunk)
                def _():
                    issue_load(i2 + 2, 1)

                @pl.when(i2 + 1 < nchunk)
                def _():
                    wait_load(0)
                    issue_gather(0)

        @pl.when(cid == 0)
        def _():
            run(yc_hbm, 0)

        @pl.when(cid == 1)
        def _():
            run(yd_hbm, 1)

        plsc.subcore_barrier()
        pltpu.sync_copy(osh.at[pl.ds(sid * RPT, RPT)],
                        acc_hbm.at[cid, pl.ds(sid * RPT, RPT)])

    return k(yc, yd, rcpack, abpack, jnp.zeros((NP, H), jnp.float32))


def _dot(x, w):
    return jnp.dot(x, w, preferred_element_type=jnp.float32)


def _full(shape):
    return pl.BlockSpec(shape, lambda i: tuple(0 for _ in shape))


def _rows(width):
    return pl.BlockSpec((RB, width), lambda i: (i, 0))


def _tc_pre(x, w1, b1, wg, wc, wd, ws, bs):
    def body(x_ref, w1_ref, b1_ref, wg_ref, wc_ref, wd_ref, ws_ref, bs_ref,
             h_ref, xg_ref, xc_ref, xd_ref, xs_ref):
        h = jnp.maximum(_dot(x_ref[...], w1_ref[...]) + b1_ref[...], 0.0)
        h_ref[...] = h
        xg_ref[...] = _dot(h, wg_ref[...])
        xc_ref[...] = _dot(h, wc_ref[...])
        xd_ref[...] = _dot(h, wd_ref[...])
        xs_ref[...] = _dot(h, ws_ref[...]) + bs_ref[...]

    o = jax.ShapeDtypeStruct((N, H), jnp.float32)
    return pl.pallas_call(
        body,
        grid=(NB,),
        in_specs=[_rows(H), _full((H, H)), _full((1, H)), _full((H, H)),
                  _full((H, H)), _full((H, H)), _full((H, H)), _full((1, H))],
        out_specs=[_rows(H)] * 5,
        out_shape=[o] * 5,
    )(x, w1, b1, wg, wc, wd, ws, bs)


def _tc_norm(xc, xd, sp):
    sp = sp.transpose(3, 2, 0, 1).reshape(NP, 2, NC * NS)

    def body(xc_ref, xd_ref, sp_ref, yc_ref, yd_ref):
        sall = sp_ref[...]
        scon = jnp.sum(sall[:, 0, :], axis=1) + SQRT_E
        sdis = jnp.sum(sall[:, 1, :], axis=1) + SQRT_E
        yc_ref[...] = xc_ref[...] / scon[:, None]
        yd_ref[...] = xd_ref[...] / sdis[:, None]

    o = jax.ShapeDtypeStruct((N, H), jnp.float32)
    return pl.pallas_call(
        body,
        grid=(NB,),
        in_specs=[_rows(H), _rows(H),
                  pl.BlockSpec((RB, 2, NC * NS), lambda i: (i, 0, 0))],
        out_specs=[_rows(H)] * 2,
        out_shape=[o] * 2,
    )(xc, xd, sp)


def _gate_combine(acc_ref, yc_ref, yd_ref, xs_ref, gc_ref, gd_ref, gs_ref,
                  gb_ref):
    oc = acc_ref[0] + SQRT_E * yc_ref[...]
    od = acc_ref[1] + SQRT_E * yd_ref[...]
    os_ = xs_ref[...]
    lg = (_dot(oc, gc_ref[...]) + _dot(od, gd_ref[...])
          + _dot(os_, gs_ref[...]) + gb_ref[...])
    m = jnp.max(lg, axis=1, keepdims=True)
    eg = jnp.exp(lg - m)
    gate = eg / jnp.sum(eg, axis=1, keepdims=True)
    return gate[:, 0:1] * oc + gate[:, 1:2] * od + gate[:, 2:3] * os_


def _tc_post0(acc, yc, yd, xs, hres, gc, gd, gs, gb, bns, bnsh,
              wg, wc, wd, ws, bs):
    def body(acc_ref, yc_ref, yd_ref, xs_ref, hres_ref, gc_ref, gd_ref,
             gs_ref, gb_ref, bns_ref, bnsh_ref, wg_ref, wc_ref, wd_ref,
             ws_ref, bs_ref, h_ref, xg_ref, xc_ref, xd_ref, xs2_ref):
        out = _gate_combine(acc_ref, yc_ref, yd_ref, xs_ref, gc_ref, gd_ref,
                            gs_ref, gb_ref)
        h = (jnp.maximum(out * bns_ref[...] + bnsh_ref[...], 0.0)
             + hres_ref[...])
        h_ref[...] = h
        xg_ref[...] = _dot(h, wg_ref[...])
        xc_ref[...] = _dot(h, wc_ref[...])
        xd_ref[...] = _dot(h, wd_ref[...])
        xs2_ref[...] = _dot(h, ws_ref[...]) + bs_ref[...]

    o = jax.ShapeDtypeStruct((N, H), jnp.float32)
    return pl.pallas_call(
        body,
        grid=(NB,),
        in_specs=[pl.BlockSpec((NC, RB, H), lambda i: (0, i, 0)),
                  _rows(H), _rows(H), _rows(H), _rows(H),
                  _full((H, 3)), _full((H, 3)), _full((H, 3)), _full((1, 3)),
                  _full((1, H)), _full((1, H)),
                  _full((H, H)), _full((H, H)), _full((H, H)), _full((H, H)),
                  _full((1, H))],
        out_specs=[_rows(H)] * 5,
        out_shape=[o] * 5,
    )(acc, yc, yd, xs, hres, gc, gd, gs, gb, bns, bnsh, wg, wc, wd, ws, bs)


def _tc_post1(acc, yc, yd, xs, hres, gc, gd, gs, gb, wcls, bcls):
    def body(acc_ref, yc_ref, yd_ref, xs_ref, hres_ref, gc_ref, gd_ref,
             gs_ref, gb_ref, wcls_ref, bcls_ref, o_ref):
        out = _gate_combine(acc_ref, yc_ref, yd_ref, xs_ref, gc_ref, gd_ref,
                            gs_ref, gb_ref)
        h = out + hres_ref[...]
        o_ref[...] = _dot(h, wcls_ref[...]) + bcls_ref[...]

    return pl.pallas_call(
        body,
        grid=(NB,),
        in_specs=[pl.BlockSpec((NC, RB, H), lambda i: (0, i, 0)),
                  _rows(H), _rows(H), _rows(H), _rows(H),
                  _full((H, 3)), _full((H, 3)), _full((H, 3)), _full((1, 3)),
                  _full((H, C)), _full((1, C))],
        out_specs=[_rows(C)],
        out_shape=[jax.ShapeDtypeStruct((N, C), jnp.float32)],
    )(acc, yc, yd, xs, hres, gc, gd, gs, gb, wcls, bcls)[0]


def _bn_fold(p):
    inv = 1.0 / jnp.sqrt(p['rv'] + 1e-5)
    scale = p['g'] * inv
    shift = p['b'] - p['rm'] * scale
    return scale, shift


def _prep_conv(p):
    gwt = p['gate_W'].T
    return dict(
        wg=p['W_g'].T, wc=p['W_con'].T, wd=p['W_dis'].T, ws=p['W_self'].T,
        bs=p['b_self'][None, :],
        gc=gwt[:H], gd=gwt[H:2 * H], gs=gwt[2 * H:], gb=p['gate_b'][None, :],
    )


def _conv_edge_phase(xg, xc, xd, row, col, rcpack):
    ab, sp = _sc_pass1(xg, rcpack)
    yc, yd = _tc_norm(xc, xd, sp)
    nch = E // K
    abpack = lax.bitcast_convert_type(ab, jnp.int32)
    acc = _sc_pass2(yc, yd, rcpack, abpack)
    return acc, yc, yd


def kernel(x, edge_index, params):
    row = edge_index[0].astype(jnp.int32)
    col = edge_index[1].astype(jnp.int32)

    s0, sh0 = _bn_fold(params['in_bn'])
    w1 = params['in_W'].T * s0[None, :]
    b1 = (params['in_b'] * s0 + sh0)[None, :]
    c0 = _prep_conv(params['conv0'])
    c1 = _prep_conv(params['conv1'])
    bns0, bnsh0 = _bn_fold(params['bn0'])

    nch = E // K
    rcpack = jnp.stack([row.reshape(nch, K), col.reshape(nch, K)], axis=1)

    h0, xg0, xc0, xd0, xs0 = _tc_pre(x, w1, b1, c0['wg'], c0['wc'], c0['wd'],
                                     c0['ws'], c0['bs'])
    acc0, yc0, yd0 = _conv_edge_phase(xg0, xc0, xd0, row, col, rcpack)
    h1, xg1, xc1, xd1, xs1 = _tc_post0(
        acc0, yc0, yd0, xs0, h0, c0['gc'], c0['gd'], c0['gs'], c0['gb'],
        bns0[None, :], bnsh0[None, :], c1['wg'], c1['wc'], c1['wd'], c1['ws'],
        c1['bs'])
    acc1, yc1, yd1 = _conv_edge_phase(xg1, xc1, xd1, row, col, rcpack)
    return _tc_post1(acc1, yc1, yd1, xs1, h1, c1['gc'], c1['gd'], c1['gs'],
                     c1['gb'], params['cls_W'].T,
                     params['cls_b'][None, :])
